# baseline (speedup 1.0000x reference)
import functools

import jax
import jax.numpy as jnp
from jax import lax
from jax.experimental import pallas as pl
from jax.experimental.pallas import tpu as pltpu
from jax.experimental.pallas import tpu_sc as plsc

B, T, D = 16, 2048, 64
VOCAB = 1000000
N = B * T
NC, NS = 2, 16
NW = NC * NS
PER_W = N // NW
CH = 128
NCH = PER_W // CH
VPR = D // 16

SLAB = 400
NSLAB = VOCAB // SLAB

_mesh = plsc.VectorSubcoreMesh(core_axis_name="c", subcore_axis_name="s")


@functools.partial(
    pl.kernel,
    mesh=_mesh,
    out_type=jax.ShapeDtypeStruct((VOCAB, D), jnp.float32),
    scratch_types=[
        pltpu.VMEM((SLAB, D), jnp.float32),
        pltpu.VMEM((SLAB, D), jnp.float32),
        pltpu.SemaphoreType.DMA,
        pltpu.SemaphoreType.DMA,
    ],
)
def _relayout_sc(wt_hbm, lin_hbm, sb0, sb1, sem0, sem1):
    wid = lax.axis_index("s") * NC + lax.axis_index("c")
    bufs = (sb0, sb1)
    sems = (sem0, sem1)
    per_tile = (NSLAB + NW - 1) // NW

    def rd(j, b):
        return pltpu.async_copy(
            wt_hbm.at[pl.ds((j * NW + wid) * SLAB, SLAB)], bufs[b], sems[b]
        )

    rd(0, 0)

    def pair(g, _):
        for b in range(2):
            j = 2 * g + b

            @pl.when(j < per_tile)
            def _():
                @pl.when(j + 1 < per_tile)
                def _():
                    rd(j + 1, b ^ 1)

                pltpu.make_async_copy(
                    wt_hbm.at[pl.ds(0, SLAB)], bufs[b], sems[b]
                ).wait()

                @pl.when(j * NW + wid < NSLAB)
                def _():
                    pltpu.sync_copy(
                        bufs[b],
                        lin_hbm.at[pl.ds((j * NW + wid) * SLAB, SLAB)],
                    )

        return ()

    lax.fori_loop(0, (per_tile + 1) // 2, pair, ())


@functools.partial(
    pl.kernel,
    mesh=_mesh,
    compiler_params=pltpu.CompilerParams(use_tc_tiling_on_sc=False),
    out_type=jax.ShapeDtypeStruct((N, D), jnp.float32),
    scratch_types=[
        pltpu.VMEM((PER_W,), jnp.int32),
        pltpu.VMEM((CH, D), jnp.float32),
        pltpu.VMEM((CH, D), jnp.float32),
        pltpu.VMEM((CH, D), jnp.float32),
        pltpu.SemaphoreType.DMA,
        pltpu.SemaphoreType.DMA,
    ],
)
def _gather_sc(x_hbm, lin_hbm, pt_hbm, out_hbm, idx_v, rb0, rb1,
               pos_v, sem0, sem1):
    wid = lax.axis_index("s") * NC + lax.axis_index("c")
    base = wid * PER_W
    pltpu.sync_copy(x_hbm.at[pl.ds(base, PER_W)], idx_v)

    bufs = (rb0, rb1)
    sems = (sem0, sem1)

    def gather(i, b):
        return pltpu.async_copy(
            lin_hbm.at[idx_v.at[pl.ds(i * CH, CH)]], bufs[b], sems[b]
        )

    gather(0, 0)

    def pair(g, _):
        for b in range(2):
            i = 2 * g + b

            @pl.when(i + 1 < NCH)
            def _():
                gather(i + 1, b ^ 1)

            pltpu.make_async_copy(
                lin_hbm.at[idx_v.at[pl.ds(i * CH, CH)]], bufs[b], sems[b]
            ).wait()
            pltpu.sync_copy(
                pt_hbm.at[pl.ds(lax.rem(base + i * CH, T), CH)], pos_v
            )
            rb = bufs[b]

            def row(r, _):
                for c in range(VPR):
                    sl = pl.ds(c * 16, 16)
                    rb[r, sl] = rb[r, sl] + pos_v[r, sl]
                return ()

            lax.fori_loop(0, CH, row, ())
            pltpu.sync_copy(rb, out_hbm.at[pl.ds(base + i * CH, CH)])
        return ()

    lax.fori_loop(0, NCH // 2, pair, ())


def kernel(x, word_table, pos_table):
    lin = _relayout_sc(word_table)
    flat = _gather_sc(x.reshape(N).astype(jnp.int32), lin, pos_table)
    return flat.reshape(B, T, D)


# speedup vs baseline: 2.7043x; 2.7043x over previous
import functools

import jax
import jax.numpy as jnp
from jax import lax
from jax.experimental import pallas as pl
from jax.experimental.pallas import tpu as pltpu
from jax.experimental.pallas import tpu_sc as plsc

B, T, D = 16, 2048, 64
N = B * T
NC, NS = 2, 16
NW = NC * NS
PER_W = N // NW
CH = 64
NCH = PER_W // CH
VPR = D // 16

_mesh = plsc.VectorSubcoreMesh(core_axis_name="c", subcore_axis_name="s")


@functools.partial(
    pl.kernel,
    mesh=_mesh,
    out_type=jax.ShapeDtypeStruct((N, D), jnp.float32),
    scratch_types=[
        pltpu.VMEM((PER_W,), jnp.int32),
        pltpu.VMEM((CH, D), jnp.float32),
        pltpu.VMEM((CH, D), jnp.float32),
        pltpu.VMEM((CH, D), jnp.float32),
        pltpu.VMEM((CH, D), jnp.float32),
        pltpu.SemaphoreType.DMA,
        pltpu.SemaphoreType.DMA,
    ],
)
def _embed_sc(x_hbm, wt_hbm, pt_hbm, out_hbm, idx_v, rb0, rb1,
              pos_v, out_v, sem0, sem1):
    wid = lax.axis_index("s") * NC + lax.axis_index("c")
    base = wid * PER_W
    pltpu.sync_copy(x_hbm.at[pl.ds(base, PER_W)], idx_v)

    bufs = (rb0, rb1)
    sems = (sem0, sem1)

    def fire_chunk(i, b):
        for g in range(CH // 16):
            v16 = idx_v[pl.ds(i * CH + g * 16, 16)]
            for l in range(16):
                k = g * 16 + l
                pltpu.async_copy(
                    wt_hbm.at[pl.ds(v16[l], 1)],
                    bufs[b].at[pl.ds(k, 1)],
                    sems[b],
                )

    fire_chunk(0, 0)

    def pair(g, _):
        for b in range(2):
            i = 2 * g + b

            @pl.when(i + 1 < NCH)
            def _():
                fire_chunk(i + 1, b ^ 1)

            pltpu.make_async_copy(
                wt_hbm.at[pl.ds(0, CH)], bufs[b], sems[b]
            ).wait()
            pltpu.sync_copy(
                pt_hbm.at[pl.ds(lax.rem(base + i * CH, T), CH)], pos_v
            )
            rb = bufs[b]
            for r in range(CH):
                for c in range(VPR):
                    sl = pl.ds(c * 16, 16)
                    out_v[r, sl] = rb[r, sl] + pos_v[r, sl]
            pltpu.sync_copy(out_v, out_hbm.at[pl.ds(base + i * CH, CH)])
        return ()

    lax.fori_loop(0, NCH // 2, pair, ())


def kernel(x, word_table, pos_table):
    flat = _embed_sc(x.reshape(N).astype(jnp.int32), word_table, pos_table)
    return flat.reshape(B, T, D)
